# baseline (speedup 1.0000x reference)
import dataclasses
import functools
import math

import jax
import jax.numpy as jnp
from jax import lax
from jax.experimental import pallas as pl
from jax.experimental.pallas import tpu as pltpu
from jax.experimental.pallas import tpu_sc as plsc

N_IMG = 4
A, H, W = 3, 52, 52
NA = A * H * W
NPAD = 8192
PRE = 6000
SPAD = 6016
POST = 1000
OPAD = 1024
NMS_TH = 0.45
IMG_SZ = 416.0
BBOX_CLIP = math.log(1000.0 / 16)
T = 128
NT = SPAD // T
LN = 16
NUNIT = N_IMG * 5


def _sc_compiler_params():
    cp = pltpu.CompilerParams()
    if "needs_layout_passes" in pltpu.CompilerParams.__dataclass_fields__:
        cp = dataclasses.replace(cp, needs_layout_passes=False)
    return cp


def _stage_a_body(obj_ref, reg_ref, anc_ref, planes_ref, rank_ref, score_ref):
    score = jax.nn.sigmoid(obj_ref[...])
    score_ref[...] = score

    ax1 = anc_ref[:, 0, :]
    ay1 = anc_ref[:, 1, :]
    ax2 = anc_ref[:, 2, :]
    ay2 = anc_ref[:, 3, :]
    dx = reg_ref[:, 0, :]
    dy = reg_ref[:, 1, :]
    dw = reg_ref[:, 2, :]
    dh = reg_ref[:, 3, :]
    widths = ax2 - ax1 + 1.0
    heights = ay2 - ay1 + 1.0
    cx = ax1 + 0.5 * widths
    cy = ay1 + 0.5 * heights
    dw = jnp.minimum(dw, BBOX_CLIP)
    dh = jnp.minimum(dh, BBOX_CLIP)
    pcx = dx * widths + cx
    pcy = dy * heights + cy
    pw = jnp.exp(dw) * widths
    ph = jnp.exp(dh) * heights
    planes_ref[:, 0, :] = jnp.clip(pcx - 0.5 * (pw - 1), 0.0, IMG_SZ - 1)
    planes_ref[:, 1, :] = jnp.clip(pcy - 0.5 * (ph - 1), 0.0, IMG_SZ - 1)
    planes_ref[:, 2, :] = jnp.clip(pcx + 0.5 * (pw - 1), 0.0, IMG_SZ - 1)
    planes_ref[:, 3, :] = jnp.clip(pcy + 0.5 * (ph - 1), 0.0, IMG_SZ - 1)
    planes_ref[:, 4, :] = score

    EBS, JBS = 128, 1024
    CPB = JBS // EBS

    def eb_body(e, _):
        se = score_ref[:, pl.ds(e * EBS, EBS)]
        seb = jnp.broadcast_to(se[:, :, None], (N_IMG, EBS, JBS))
        ide = lax.broadcasted_iota(jnp.int32, (1, EBS, 1), 1) + e * EBS

        def jb_body(j, acc):
            sj = score_ref[:, pl.ds(j * JBS, JBS)][:, None, :]
            idj = lax.broadcasted_iota(jnp.int32, (1, 1, JBS), 2) + j * JBS
            cmp = (sj > seb) | ((sj == seb) & (idj < ide))
            return acc + cmp.astype(jnp.int32)

        acc = lax.fori_loop(0, NPAD // JBS, jb_body,
                            jnp.zeros((N_IMG, EBS, JBS), jnp.int32))
        rank_ref[:, pl.ds(e * EBS, EBS)] = jnp.sum(acc, axis=2)
        return 0

    lax.fori_loop(0, NPAD // EBS, eb_body, 0)


def _stage_a(obj, reg, anc):
    return pl.pallas_call(
        _stage_a_body,
        out_shape=[
            jax.ShapeDtypeStruct((N_IMG, 5, NPAD), jnp.float32),
            jax.ShapeDtypeStruct((N_IMG, NPAD), jnp.int32),
        ],
        scratch_shapes=[pltpu.VMEM((N_IMG, NPAD), jnp.float32)],
    )(obj, reg, anc)


def _stage_b(planes20, rank):
    mesh = plsc.VectorSubcoreMesh(core_axis_name="c", subcore_axis_name="s")

    @functools.partial(
        pl.kernel, mesh=mesh,
        out_type=jax.ShapeDtypeStruct((NUNIT, SPAD), jnp.float32),
        scratch_types=[
            pltpu.VMEM((NPAD,), jnp.float32),
            pltpu.VMEM((NPAD,), jnp.int32),
            pltpu.VMEM((SPAD,), jnp.float32),
        ],
        compiler_params=_sc_compiler_params(),
    )
    def sb_kernel(planes_hbm, rank_hbm, out_hbm, vals_v, rank_v, out_v):
        wid = lax.axis_index("c") * 16 + lax.axis_index("s")

        @pl.when(wid < NUNIT)
        def _():
            pltpu.sync_copy(planes_hbm.at[wid], vals_v)
            pltpu.sync_copy(rank_hbm.at[wid // 5], rank_v)

            @pl.loop(0, NPAD // LN)
            def _(i):
                r = rank_v[pl.ds(i * LN, LN)]
                v = vals_v[pl.ds(i * LN, LN)]
                msk = r < SPAD
                rc = jnp.minimum(r, SPAD - 1)
                plsc.store_scatter(out_v, [rc], v, mask=msk)

            pltpu.sync_copy(out_v, out_hbm.at[wid])

    return sb_kernel(planes20, rank)


def _iou_block(x1a, y1a, x2a, y2a, aa, x1b, y1b, x2b, y2b, ab):
    ltx = jnp.maximum(x1a[:, :, None], x1b[:, None, :])
    lty = jnp.maximum(y1a[:, :, None], y1b[:, None, :])
    rbx = jnp.minimum(x2a[:, :, None], x2b[:, None, :])
    rby = jnp.minimum(y2a[:, :, None], y2b[:, None, :])
    wx = jnp.maximum(rbx - ltx + 1.0, 0.0)
    wy = jnp.maximum(rby - lty + 1.0, 0.0)
    inter = wx * wy
    return inter / ((aa[:, :, None] + ab[:, None, :]) - inter)


def _outer(u, v):
    return lax.dot_general(u.astype(jnp.bfloat16), v.astype(jnp.bfloat16),
                           (((1,), (1,)), ((0,), (0,))),
                           preferred_element_type=jnp.float32)


def _stage_c_body(srt_ref, keep_ref, crank_ref, alive_ref, area_ref, m_ref,
                  eff_ref):
    SENT = 2.0e9
    x1 = srt_ref[:, 0, :]
    y1 = srt_ref[:, 1, :]
    x2 = srt_ref[:, 2, :]
    y2 = srt_ref[:, 3, :]
    area_ref[...] = (x2 - x1 + 1.0) * (y2 - y1 + 1.0)
    pos = lax.broadcasted_iota(jnp.int32, (N_IMG, SPAD), 1)
    alive_ref[...] = (pos < PRE).astype(jnp.float32)

    ones_l = jnp.ones((N_IMG, 1, T), jnp.float32)

    def tile_body(t, _):
        tb = t * T
        xt1 = srt_ref[:, 0, pl.ds(tb, T)]
        yt1 = srt_ref[:, 1, pl.ds(tb, T)]
        xt2 = srt_ref[:, 2, pl.ds(tb, T)]
        yt2 = srt_ref[:, 3, pl.ds(tb, T)]
        at = area_ref[:, pl.ds(tb, T)]
        sh = (N_IMG, T, T)
        xt1b = jnp.broadcast_to(xt1[:, :, None], sh)
        yt1b = jnp.broadcast_to(yt1[:, :, None], sh)
        xt2b = jnp.broadcast_to(xt2[:, :, None], sh)
        yt2b = jnp.broadcast_to(yt2[:, :, None], sh)
        atb = jnp.broadcast_to(at[:, :, None], sh)

        def pref_body(s, acc):
            sb = s * T
            xp1 = eff_ref[:, 0, pl.ds(sb, T)][:, None, :]
            yp1 = eff_ref[:, 1, pl.ds(sb, T)][:, None, :]
            xp2 = eff_ref[:, 2, pl.ds(sb, T)][:, None, :]
            yp2 = eff_ref[:, 3, pl.ds(sb, T)][:, None, :]
            ap = eff_ref[:, 4, pl.ds(sb, T)][:, None, :]
            wx = jnp.maximum(jnp.minimum(xt2b, xp2) - jnp.maximum(xt1b, xp1)
                             + 1.0, 0.0)
            wy = jnp.maximum(jnp.minimum(yt2b, yp2) - jnp.maximum(yt1b, yp1)
                             + 1.0, 0.0)
            inter = wx * wy
            iou = inter / ((atb + ap) - inter)
            return jnp.maximum(acc, iou)

        acc = lax.fori_loop(0, t, pref_body, jnp.zeros(sh, jnp.float32))
        dead = (jnp.max(acc, axis=2) > NMS_TH).astype(jnp.float32)
        alt0 = alive_ref[:, pl.ds(tb, T)] * (1.0 - dead)

        iou_tt = _iou_block(xt1, yt1, xt2, yt2, at, xt1, yt1, xt2, yt2, at)
        tri = (lax.broadcasted_iota(jnp.int32, (1, T, T), 2)
               > lax.broadcasted_iota(jnp.int32, (1, T, T), 1))
        m_ref[...] = (((iou_tt > NMS_TH) & tri).astype(jnp.float32)
                      * _outer(alt0[:, None, :], ones_l))

        s0 = jnp.sum(m_ref[...])

        def fix_cond(c):
            return c[1] < c[0]

        def fix_body(c):
            mat = m_ref[...]
            inc = jnp.max(mat, axis=1, keepdims=True)
            can = (inc == 0.0).astype(jnp.float32)
            can_full = _outer(can, ones_l)
            defsup = jnp.max(mat * can_full, axis=1, keepdims=True)
            kill_full = _outer(defsup, ones_l)
            new = mat * (1.0 - kill_full)
            m_ref[...] = new
            return (c[1], jnp.sum(new))

        lax.while_loop(fix_cond, fix_body, (s0 + 1.0, s0))
        suppressed = jnp.max(m_ref[...], axis=1)
        alt = alt0 * (1.0 - suppressed)
        alive_ref[:, pl.ds(tb, T)] = alt

        live = alt > 0
        eff_ref[:, 0, pl.ds(tb, T)] = jnp.where(live, xt1, SENT)
        eff_ref[:, 1, pl.ds(tb, T)] = jnp.where(live, yt1, SENT)
        eff_ref[:, 2, pl.ds(tb, T)] = jnp.where(live, xt2, SENT)
        eff_ref[:, 3, pl.ds(tb, T)] = jnp.where(live, yt2, SENT)
        eff_ref[:, 4, pl.ds(tb, T)] = jnp.where(live, at, 1.0)
        return 0

    lax.fori_loop(0, NT, tile_body, 0)

    tri_cs = (lax.broadcasted_iota(jnp.int32, (T, T), 0)
              <= lax.broadcasted_iota(jnp.int32, (T, T), 1)).astype(jnp.bfloat16)

    def csum_body(t, off):
        k = alive_ref[:, pl.ds(t * T, T)]
        cs = lax.dot_general(k.astype(jnp.bfloat16), tri_cs,
                             (((1,), (0,)), ((), ())),
                             preferred_element_type=jnp.float32)
        crank_ref[:, pl.ds(t * T, T)] = (cs + off - 1.0).astype(jnp.int32)
        keep_ref[:, pl.ds(t * T, T)] = k.astype(jnp.int32)
        return off + jnp.sum(k, axis=1, keepdims=True)

    lax.fori_loop(0, NT, csum_body, jnp.zeros((N_IMG, 1), jnp.float32))


def _stage_c(srt):
    return pl.pallas_call(
        _stage_c_body,
        out_shape=[
            jax.ShapeDtypeStruct((N_IMG, SPAD), jnp.int32),
            jax.ShapeDtypeStruct((N_IMG, SPAD), jnp.int32),
        ],
        scratch_shapes=[
            pltpu.VMEM((N_IMG, SPAD), jnp.float32),
            pltpu.VMEM((N_IMG, SPAD), jnp.float32),
            pltpu.VMEM((N_IMG, T, T), jnp.float32),
            pltpu.VMEM((N_IMG, 5, SPAD), jnp.float32),
        ],
    )(srt)


def _stage_d(srt20, keep, crank):
    mesh = plsc.VectorSubcoreMesh(core_axis_name="c", subcore_axis_name="s")

    @functools.partial(
        pl.kernel, mesh=mesh,
        out_type=jax.ShapeDtypeStruct((NUNIT, OPAD), jnp.float32),
        scratch_types=[
            pltpu.VMEM((SPAD,), jnp.float32),
            pltpu.VMEM((SPAD,), jnp.int32),
            pltpu.VMEM((SPAD,), jnp.int32),
            pltpu.VMEM((OPAD,), jnp.float32),
        ],
        compiler_params=_sc_compiler_params(),
    )
    def sd_kernel(srt_hbm, keep_hbm, crank_hbm, out_hbm,
                  vals_v, keep_v, crank_v, out_v):
        wid = lax.axis_index("c") * 16 + lax.axis_index("s")

        @pl.when(wid < NUNIT)
        def _():
            pltpu.sync_copy(srt_hbm.at[wid], vals_v)
            pltpu.sync_copy(keep_hbm.at[wid // 5], keep_v)
            pltpu.sync_copy(crank_hbm.at[wid // 5], crank_v)

            @pl.loop(0, OPAD // LN)
            def _(i):
                out_v[pl.ds(i * LN, LN)] = jnp.zeros((LN,), jnp.float32)

            @pl.loop(0, SPAD // LN)
            def _(i):
                k = keep_v[pl.ds(i * LN, LN)]
                c = crank_v[pl.ds(i * LN, LN)]
                v = vals_v[pl.ds(i * LN, LN)]
                msk = (k > 0) & (c < POST)
                cc = jnp.minimum(jnp.maximum(c, 0), OPAD - 1)
                plsc.store_scatter(out_v, [cc], v, mask=msk)

            pltpu.sync_copy(out_v, out_hbm.at[wid])

    return sd_kernel(srt20, keep, crank)


def kernel(objectness, box_regression, cls, anchors):
    del cls
    n = objectness.shape[0]

    obj = jnp.transpose(objectness, (0, 2, 3, 1)).reshape(n, NA)
    obj = jnp.pad(obj, ((0, 0), (0, NPAD - NA)), constant_values=-1e4)
    reg = box_regression.reshape(n, A, 4, H * W)
    reg = jnp.transpose(reg, (0, 2, 3, 1)).reshape(n, 4, NA)
    reg = jnp.pad(reg, ((0, 0), (0, 0), (0, NPAD - NA)))
    anc = jnp.transpose(anchors.reshape(n, NA, 4), (0, 2, 1))
    anc = jnp.pad(anc, ((0, 0), (0, 0), (0, NPAD - NA)))

    planes, rank = _stage_a(obj, reg, anc)
    srt20 = _stage_b(planes.reshape(NUNIT, NPAD), rank)
    keep, crank = _stage_c(srt20.reshape(N_IMG, 5, SPAD))
    out20 = _stage_d(srt20, keep, crank)
    out = out20.reshape(n, 5, OPAD)[:, :, :POST]
    return jnp.transpose(out, (0, 2, 1))


# speedup vs baseline: 1.4541x; 1.4541x over previous
import dataclasses
import functools
import math

import jax
import jax.numpy as jnp
from jax import lax
from jax.experimental import pallas as pl
from jax.experimental.pallas import tpu as pltpu
from jax.experimental.pallas import tpu_sc as plsc

N_IMG = 4
A, H, W = 3, 52, 52
NA = A * H * W
NPAD = 8192
PRE = 6000
SPAD = 6016
POST = 1000
OPAD = 1024
NMS_TH = 0.45
IMG_SZ = 416.0
BBOX_CLIP = math.log(1000.0 / 16)
T = 128
NT = SPAD // T
LN = 16
NUNIT = N_IMG * 5


def _sc_compiler_params():
    cp = pltpu.CompilerParams()
    if "needs_layout_passes" in pltpu.CompilerParams.__dataclass_fields__:
        cp = dataclasses.replace(cp, needs_layout_passes=False)
    return cp


def _stage_a_body(obj_ref, reg_ref, anc_ref, planes_ref, rank_ref, score_ref):
    score = jax.nn.sigmoid(obj_ref[...])
    score_ref[...] = score

    ax1 = anc_ref[:, 0, :]
    ay1 = anc_ref[:, 1, :]
    ax2 = anc_ref[:, 2, :]
    ay2 = anc_ref[:, 3, :]
    dx = reg_ref[:, 0, :]
    dy = reg_ref[:, 1, :]
    dw = reg_ref[:, 2, :]
    dh = reg_ref[:, 3, :]
    widths = ax2 - ax1 + 1.0
    heights = ay2 - ay1 + 1.0
    cx = ax1 + 0.5 * widths
    cy = ay1 + 0.5 * heights
    dw = jnp.minimum(dw, BBOX_CLIP)
    dh = jnp.minimum(dh, BBOX_CLIP)
    pcx = dx * widths + cx
    pcy = dy * heights + cy
    pw = jnp.exp(dw) * widths
    ph = jnp.exp(dh) * heights
    planes_ref[:, 0, :] = jnp.clip(pcx - 0.5 * (pw - 1), 0.0, IMG_SZ - 1)
    planes_ref[:, 1, :] = jnp.clip(pcy - 0.5 * (ph - 1), 0.0, IMG_SZ - 1)
    planes_ref[:, 2, :] = jnp.clip(pcx + 0.5 * (pw - 1), 0.0, IMG_SZ - 1)
    planes_ref[:, 3, :] = jnp.clip(pcy + 0.5 * (ph - 1), 0.0, IMG_SZ - 1)
    planes_ref[:, 4, :] = score

    EBS, JBS = 128, 1024
    rank_ref[...] = jnp.zeros((N_IMG, NPAD), jnp.int32)

    def jb_body(j, _):
        sjb = jnp.broadcast_to(score_ref[:, pl.ds(j * JBS, JBS)][:, :, None],
                               (N_IMG, JBS, EBS))
        idjb = jnp.broadcast_to(
            lax.broadcasted_iota(jnp.int32, (1, JBS, 1), 1) + j * JBS,
            (1, JBS, EBS))

        def eb_body(e, _):
            se = score_ref[:, pl.ds(e * EBS, EBS)][:, None, :]
            ide = lax.broadcasted_iota(jnp.int32, (1, 1, EBS), 2) + e * EBS
            cmp = (sjb > se) | ((sjb == se) & (idjb < ide))
            rank_ref[:, pl.ds(e * EBS, EBS)] += jnp.sum(
                cmp.astype(jnp.int32), axis=1)
            return 0

        lax.fori_loop(0, NPAD // EBS, eb_body, 0)
        return 0

    lax.fori_loop(0, NPAD // JBS, jb_body, 0)


def _stage_a(obj, reg, anc):
    return pl.pallas_call(
        _stage_a_body,
        out_shape=[
            jax.ShapeDtypeStruct((N_IMG, 5, NPAD), jnp.float32),
            jax.ShapeDtypeStruct((N_IMG, NPAD), jnp.int32),
        ],
        scratch_shapes=[pltpu.VMEM((N_IMG, NPAD), jnp.float32)],
    )(obj, reg, anc)


def _stage_b(planes20, rank):
    mesh = plsc.VectorSubcoreMesh(core_axis_name="c", subcore_axis_name="s")

    @functools.partial(
        pl.kernel, mesh=mesh,
        out_type=jax.ShapeDtypeStruct((NUNIT, SPAD), jnp.float32),
        scratch_types=[
            pltpu.VMEM((NPAD,), jnp.float32),
            pltpu.VMEM((NPAD,), jnp.int32),
            pltpu.VMEM((SPAD,), jnp.float32),
        ],
        compiler_params=_sc_compiler_params(),
    )
    def sb_kernel(planes_hbm, rank_hbm, out_hbm, vals_v, rank_v, out_v):
        wid = lax.axis_index("c") * 16 + lax.axis_index("s")

        @pl.when(wid < NUNIT)
        def _():
            pltpu.sync_copy(planes_hbm.at[wid], vals_v)
            pltpu.sync_copy(rank_hbm.at[wid // 5], rank_v)

            @pl.loop(0, NPAD // LN)
            def _(i):
                r = rank_v[pl.ds(i * LN, LN)]
                v = vals_v[pl.ds(i * LN, LN)]
                msk = r < SPAD
                rc = jnp.minimum(r, SPAD - 1)
                plsc.store_scatter(out_v, [rc], v, mask=msk)

            pltpu.sync_copy(out_v, out_hbm.at[wid])

    return sb_kernel(planes20, rank)


def _iou_block(x1a, y1a, x2a, y2a, aa, x1b, y1b, x2b, y2b, ab):
    ltx = jnp.maximum(x1a[:, :, None], x1b[:, None, :])
    lty = jnp.maximum(y1a[:, :, None], y1b[:, None, :])
    rbx = jnp.minimum(x2a[:, :, None], x2b[:, None, :])
    rby = jnp.minimum(y2a[:, :, None], y2b[:, None, :])
    wx = jnp.maximum(rbx - ltx + 1.0, 0.0)
    wy = jnp.maximum(rby - lty + 1.0, 0.0)
    inter = wx * wy
    return inter / ((aa[:, :, None] + ab[:, None, :]) - inter)


def _outer(u, v):
    return lax.dot_general(u.astype(jnp.bfloat16), v.astype(jnp.bfloat16),
                           (((1,), (1,)), ((0,), (0,))),
                           preferred_element_type=jnp.float32)


def _stage_c_body(srt_ref, keep_ref, crank_ref, alive_ref, area_ref, m_ref,
                  eff_ref):
    SENT = 2.0e9
    x1 = srt_ref[:, 0, :]
    y1 = srt_ref[:, 1, :]
    x2 = srt_ref[:, 2, :]
    y2 = srt_ref[:, 3, :]
    area_ref[...] = (x2 - x1 + 1.0) * (y2 - y1 + 1.0)
    pos = lax.broadcasted_iota(jnp.int32, (N_IMG, SPAD), 1)
    alive_ref[...] = (pos < PRE).astype(jnp.float32)

    ones_l = jnp.ones((N_IMG, 1, T), jnp.float32)

    def tile_body(t, _):
        tb = t * T
        xt1 = srt_ref[:, 0, pl.ds(tb, T)]
        yt1 = srt_ref[:, 1, pl.ds(tb, T)]
        xt2 = srt_ref[:, 2, pl.ds(tb, T)]
        yt2 = srt_ref[:, 3, pl.ds(tb, T)]
        at = area_ref[:, pl.ds(tb, T)]
        sh = (N_IMG, T, T)
        xt1b = jnp.broadcast_to(xt1[:, :, None], sh)
        yt1b = jnp.broadcast_to(yt1[:, :, None], sh)
        xt2b = jnp.broadcast_to(xt2[:, :, None], sh)
        yt2b = jnp.broadcast_to(yt2[:, :, None], sh)
        atb = jnp.broadcast_to(at[:, :, None], sh)

        def pref_body(s, acc):
            sb = s * T
            xp1 = eff_ref[:, 0, pl.ds(sb, T)][:, None, :]
            yp1 = eff_ref[:, 1, pl.ds(sb, T)][:, None, :]
            xp2 = eff_ref[:, 2, pl.ds(sb, T)][:, None, :]
            yp2 = eff_ref[:, 3, pl.ds(sb, T)][:, None, :]
            ap = eff_ref[:, 4, pl.ds(sb, T)][:, None, :]
            wx = jnp.maximum(jnp.minimum(xt2b, xp2) - jnp.maximum(xt1b, xp1)
                             + 1.0, 0.0)
            wy = jnp.maximum(jnp.minimum(yt2b, yp2) - jnp.maximum(yt1b, yp1)
                             + 1.0, 0.0)
            inter = wx * wy
            iou = inter / ((atb + ap) - inter)
            return jnp.maximum(acc, iou)

        acc = lax.fori_loop(0, t, pref_body, jnp.zeros(sh, jnp.float32))
        dead = (jnp.max(acc, axis=2) > NMS_TH).astype(jnp.float32)
        alt0 = alive_ref[:, pl.ds(tb, T)] * (1.0 - dead)

        iou_tt = _iou_block(xt1, yt1, xt2, yt2, at, xt1, yt1, xt2, yt2, at)
        tri = (lax.broadcasted_iota(jnp.int32, (1, T, T), 2)
               > lax.broadcasted_iota(jnp.int32, (1, T, T), 1))
        m_ref[...] = (((iou_tt > NMS_TH) & tri).astype(jnp.float32)
                      * _outer(alt0[:, None, :], ones_l))

        s0 = jnp.sum(m_ref[...])

        def fix_cond(c):
            return c[1] < c[0]

        def fix_body(c):
            mat = m_ref[...]
            inc = jnp.max(mat, axis=1, keepdims=True)
            can = (inc == 0.0).astype(jnp.float32)
            can_full = _outer(can, ones_l)
            defsup = jnp.max(mat * can_full, axis=1, keepdims=True)
            kill_full = _outer(defsup, ones_l)
            new = mat * (1.0 - kill_full)
            m_ref[...] = new
            return (c[1], jnp.sum(new))

        lax.while_loop(fix_cond, fix_body, (s0 + 1.0, s0))
        suppressed = jnp.max(m_ref[...], axis=1)
        alt = alt0 * (1.0 - suppressed)
        alive_ref[:, pl.ds(tb, T)] = alt

        live = alt > 0
        eff_ref[:, 0, pl.ds(tb, T)] = jnp.where(live, xt1, SENT)
        eff_ref[:, 1, pl.ds(tb, T)] = jnp.where(live, yt1, SENT)
        eff_ref[:, 2, pl.ds(tb, T)] = jnp.where(live, xt2, SENT)
        eff_ref[:, 3, pl.ds(tb, T)] = jnp.where(live, yt2, SENT)
        eff_ref[:, 4, pl.ds(tb, T)] = jnp.where(live, at, 1.0)
        return 0

    lax.fori_loop(0, NT, tile_body, 0)

    tri_cs = (lax.broadcasted_iota(jnp.int32, (T, T), 0)
              <= lax.broadcasted_iota(jnp.int32, (T, T), 1)).astype(jnp.bfloat16)

    def csum_body(t, off):
        k = alive_ref[:, pl.ds(t * T, T)]
        cs = lax.dot_general(k.astype(jnp.bfloat16), tri_cs,
                             (((1,), (0,)), ((), ())),
                             preferred_element_type=jnp.float32)
        crank_ref[:, pl.ds(t * T, T)] = (cs + off - 1.0).astype(jnp.int32)
        keep_ref[:, pl.ds(t * T, T)] = k.astype(jnp.int32)
        return off + jnp.sum(k, axis=1, keepdims=True)

    lax.fori_loop(0, NT, csum_body, jnp.zeros((N_IMG, 1), jnp.float32))


def _stage_c(srt):
    return pl.pallas_call(
        _stage_c_body,
        out_shape=[
            jax.ShapeDtypeStruct((N_IMG, SPAD), jnp.int32),
            jax.ShapeDtypeStruct((N_IMG, SPAD), jnp.int32),
        ],
        scratch_shapes=[
            pltpu.VMEM((N_IMG, SPAD), jnp.float32),
            pltpu.VMEM((N_IMG, SPAD), jnp.float32),
            pltpu.VMEM((N_IMG, T, T), jnp.float32),
            pltpu.VMEM((N_IMG, 5, SPAD), jnp.float32),
        ],
    )(srt)


def _stage_d(srt20, keep, crank):
    mesh = plsc.VectorSubcoreMesh(core_axis_name="c", subcore_axis_name="s")

    @functools.partial(
        pl.kernel, mesh=mesh,
        out_type=jax.ShapeDtypeStruct((NUNIT, OPAD), jnp.float32),
        scratch_types=[
            pltpu.VMEM((SPAD,), jnp.float32),
            pltpu.VMEM((SPAD,), jnp.int32),
            pltpu.VMEM((SPAD,), jnp.int32),
            pltpu.VMEM((OPAD,), jnp.float32),
        ],
        compiler_params=_sc_compiler_params(),
    )
    def sd_kernel(srt_hbm, keep_hbm, crank_hbm, out_hbm,
                  vals_v, keep_v, crank_v, out_v):
        wid = lax.axis_index("c") * 16 + lax.axis_index("s")

        @pl.when(wid < NUNIT)
        def _():
            pltpu.sync_copy(srt_hbm.at[wid], vals_v)
            pltpu.sync_copy(keep_hbm.at[wid // 5], keep_v)
            pltpu.sync_copy(crank_hbm.at[wid // 5], crank_v)

            @pl.loop(0, OPAD // LN)
            def _(i):
                out_v[pl.ds(i * LN, LN)] = jnp.zeros((LN,), jnp.float32)

            @pl.loop(0, SPAD // LN)
            def _(i):
                k = keep_v[pl.ds(i * LN, LN)]
                c = crank_v[pl.ds(i * LN, LN)]
                v = vals_v[pl.ds(i * LN, LN)]
                msk = (k > 0) & (c < POST)
                cc = jnp.minimum(jnp.maximum(c, 0), OPAD - 1)
                plsc.store_scatter(out_v, [cc], v, mask=msk)

            pltpu.sync_copy(out_v, out_hbm.at[wid])

    return sd_kernel(srt20, keep, crank)


def kernel(objectness, box_regression, cls, anchors):
    del cls
    n = objectness.shape[0]

    obj = jnp.transpose(objectness, (0, 2, 3, 1)).reshape(n, NA)
    obj = jnp.pad(obj, ((0, 0), (0, NPAD - NA)), constant_values=-1e4)
    reg = box_regression.reshape(n, A, 4, H * W)
    reg = jnp.transpose(reg, (0, 2, 3, 1)).reshape(n, 4, NA)
    reg = jnp.pad(reg, ((0, 0), (0, 0), (0, NPAD - NA)))
    anc = jnp.transpose(anchors.reshape(n, NA, 4), (0, 2, 1))
    anc = jnp.pad(anc, ((0, 0), (0, 0), (0, NPAD - NA)))

    planes, rank = _stage_a(obj, reg, anc)
    srt20 = _stage_b(planes.reshape(NUNIT, NPAD), rank)
    keep, crank = _stage_c(srt20.reshape(N_IMG, 5, SPAD))
    out20 = _stage_d(srt20, keep, crank)
    out = out20.reshape(n, 5, OPAD)[:, :, :POST]
    return jnp.transpose(out, (0, 2, 1))
